# baseline (speedup 1.0000x reference)
import functools

import jax
import jax.numpy as jnp
from jax import lax
from jax.experimental import pallas as pl
from jax.experimental.pallas import tpu as pltpu
from jax.experimental.pallas import tpu_sc as plsc

_N = 1024
_K = 64
_TB = 512

_info = plsc.get_sparse_core_info()
_NC, _NS = _info.num_cores, _info.num_subcores
_NW = _NC * _NS


def _argmin_block(x_ref, cb_ref, idx_ref, cbt_ref):
    i = pl.program_id(0)
    x = x_ref[...]
    cb = cb_ref[...]
    sim = jnp.dot(x, cb, preferred_element_type=jnp.float32)
    xsq = jnp.sum(x * x, axis=1, keepdims=True)
    csq = jnp.sum(cb * cb, axis=0, keepdims=True)
    dist = xsq + csq - 2.0 * sim
    m = jnp.min(dist, axis=1, keepdims=True)
    ids = jax.lax.broadcasted_iota(jnp.int32, (_TB, _N), 1)
    idx_ref[...] = jnp.min(jnp.where(dist == m, ids, _N), axis=1)

    @pl.when(i == 0)
    def _():
        cbt_ref[...] = cb.T


def _tc_argmin(flat, codebook):
    t = flat.shape[0]
    grid = t // _TB
    return pl.pallas_call(
        _argmin_block,
        grid=(grid,),
        in_specs=[
            pl.BlockSpec((_TB, _K), lambda i: (i, 0)),
            pl.BlockSpec((_K, _N), lambda i: (0, 0)),
        ],
        out_specs=[
            pl.BlockSpec((_TB,), lambda i: (i,)),
            pl.BlockSpec((_N, _K), lambda i: (0, 0)),
        ],
        out_shape=[
            jax.ShapeDtypeStruct((t,), jnp.int32),
            jax.ShapeDtypeStruct((_N, _K), jnp.float32),
        ],
    )(flat, codebook)


def _sc_gather(table, idx, t):
    bpw = t // _NW
    mesh = plsc.VectorSubcoreMesh(core_axis_name="c", subcore_axis_name="s")

    @functools.partial(
        pl.kernel, mesh=mesh,
        compiler_params=pltpu.CompilerParams(use_tc_tiling_on_sc=False),
        out_type=jax.ShapeDtypeStruct((t, _K), jnp.float32),
        scratch_types=[
            pltpu.VMEM((bpw,), jnp.int32),
            pltpu.VMEM((bpw, _K), jnp.float32),
            pltpu.SemaphoreType.DMA,
        ],
    )
    def gather_kernel(table_hbm, idx_hbm, out_hbm, idx_v, rows_v, sem):
        wid = lax.axis_index("s") * _NC + lax.axis_index("c")
        base = wid * bpw
        pltpu.sync_copy(idx_hbm.at[pl.ds(base, bpw)], idx_v)
        pltpu.async_copy(table_hbm.at[idx_v], rows_v, sem).wait()
        pltpu.sync_copy(rows_v, out_hbm.at[pl.ds(base, bpw)])

    return gather_kernel(table, idx)


def kernel(z, codebook):
    shape = z.shape
    flat = z.reshape(-1, _K)
    t = flat.shape[0]
    idx, cbt = _tc_argmin(flat, codebook)
    out = _sc_gather(cbt, idx, t)
    return out.reshape(shape)


# speedup vs baseline: 1.2805x; 1.2805x over previous
import functools

import jax
import jax.numpy as jnp
from jax import lax
from jax.experimental import pallas as pl
from jax.experimental.pallas import tpu as pltpu
from jax.experimental.pallas import tpu_sc as plsc

_N = 1024
_K = 64
_TB = 1024

_info = plsc.get_sparse_core_info()
_NC, _NS = _info.num_cores, _info.num_subcores
_NW = _NC * _NS


def _argmin_block(x_ref, cb_ref, idx_ref, cbt_ref):
    i = pl.program_id(0)
    x = x_ref[...]
    cb = cb_ref[...]
    sim2 = jnp.dot(x + x, cb, preferred_element_type=jnp.float32)
    xsq = jnp.sum(x * x, axis=1, keepdims=True)
    csq = jnp.sum(cb * cb, axis=0, keepdims=True)
    dist = (xsq + csq) - sim2
    idx_ref[...] = jnp.argmin(dist, axis=1).astype(jnp.int32)

    @pl.when(i == 0)
    def _():
        cbt_ref[...] = cb.T


def _tc_argmin(flat, codebook):
    t = flat.shape[0]
    grid = t // _TB
    return pl.pallas_call(
        _argmin_block,
        grid=(grid,),
        in_specs=[
            pl.BlockSpec((_TB, _K), lambda i: (i, 0)),
            pl.BlockSpec((_K, _N), lambda i: (0, 0)),
        ],
        out_specs=[
            pl.BlockSpec((_TB,), lambda i: (i,)),
            pl.BlockSpec((_N, _K), lambda i: (0, 0)),
        ],
        out_shape=[
            jax.ShapeDtypeStruct((t,), jnp.int32),
            jax.ShapeDtypeStruct((_N, _K), jnp.float32),
        ],
    )(flat, codebook)


def _sc_gather(table, idx, t):
    bpw = t // _NW
    mesh = plsc.VectorSubcoreMesh(core_axis_name="c", subcore_axis_name="s")

    @functools.partial(
        pl.kernel, mesh=mesh,
        compiler_params=pltpu.CompilerParams(use_tc_tiling_on_sc=False),
        out_type=jax.ShapeDtypeStruct((t, _K), jnp.float32),
        scratch_types=[
            pltpu.VMEM((bpw,), jnp.int32),
            pltpu.VMEM((bpw, _K), jnp.float32),
            pltpu.SemaphoreType.DMA,
        ],
    )
    def gather_kernel(table_hbm, idx_hbm, out_hbm, idx_v, rows_v, sem):
        wid = lax.axis_index("s") * _NC + lax.axis_index("c")
        base = wid * bpw
        pltpu.sync_copy(idx_hbm.at[pl.ds(base, bpw)], idx_v)
        pltpu.async_copy(table_hbm.at[idx_v], rows_v, sem).wait()
        pltpu.sync_copy(rows_v, out_hbm.at[pl.ds(base, bpw)])

    return gather_kernel(table, idx)


def kernel(z, codebook):
    shape = z.shape
    flat = z.reshape(-1, _K)
    t = flat.shape[0]
    idx, cbt = _tc_argmin(flat, codebook)
    out = _sc_gather(cbt, idx, t)
    return out.reshape(shape)
